# baseline (speedup 1.0000x reference)
import functools

import jax
import jax.numpy as jnp
from jax import lax
from jax.experimental import pallas as pl
from jax.experimental.pallas import tpu as pltpu
from jax.experimental.pallas import tpu_sc as plsc

_NC = 2
_NS = 16
_DW = 16
_K = 40
_G = 5


def _mesh():
    return plsc.VectorSubcoreMesh(
        core_axis_name="c", subcore_axis_name="s",
        num_cores=_NC, num_subcores=_NS)


def _striped_copy(n, s, copy_fn):
    rpt8 = (n // _NS) // 8 * 8
    rem = n - _NS * rpt8
    copy_fn(pl.ds(s * rpt8, rpt8))
    if rem:
        @pl.when(s == _NS - 1)
        def _():
            copy_fn(pl.ds(_NS * rpt8, rem))


def _sc_degree(dst, init):
    e = dst.shape[0]
    n = init.shape[1]
    nw = _NC * _NS
    epw = e // nw
    nwin = epw // _K

    @functools.partial(
        pl.kernel,
        out_type=jax.ShapeDtypeStruct((_NC, n, _DW), jnp.float32),
        mesh=_mesh(),
        scratch_types=[
            pltpu.VMEM((_K,), jnp.int32),
            pltpu.VMEM((_K, _DW), jnp.float32),
            pltpu.VMEM_SHARED((n, _DW), jnp.float32),
        ],
    )
    def deg_kernel(dst_hbm, init_hbm, out_hbm, dstv, ones, acc):
        c = lax.axis_index("c")
        s = lax.axis_index("s")
        base = (c * _NS + s) * epw

        @pl.loop(0, _K)
        def _(i):
            ones[i, :] = jnp.full((_DW,), 1.0, jnp.float32)

        _striped_copy(n, s, lambda sl: pltpu.sync_copy(
            init_hbm.at[c].at[sl], acc.at[sl]))
        plsc.subcore_barrier()

        @pl.loop(0, nwin)
        def _(t):
            pltpu.sync_copy(dst_hbm.at[pl.ds(base + t * _K, _K)], dstv)
            pltpu.sync_copy(ones, acc.at[dstv], add=True)

        plsc.subcore_barrier()
        _striped_copy(n, s, lambda sl: pltpu.sync_copy(
            acc.at[sl], out_hbm.at[c].at[sl]))

    return deg_kernel(dst, init)


@functools.lru_cache(maxsize=None)
def _make_sc_aggregate(n, h, e):
    nw = _NC * _NS
    epw = e // nw
    ngrp = epw // (_G * _K)

    @functools.partial(
        pl.kernel,
        out_type=jax.ShapeDtypeStruct((_NC, n, h), jnp.float32),
        mesh=_mesh(),
        scratch_types=[pltpu.VMEM((_K,), jnp.int32)] * _G
        + [pltpu.VMEM((_K,), jnp.int32)] * _G
        + [pltpu.VMEM((_K, h), jnp.float32)] * _G
        + [
            pltpu.VMEM_SHARED((n, h), jnp.float32),
            pltpu.SemaphoreType.DMA,
        ],
    )
    def agg_kernel(hs_hbm, src_hbm, dst_hbm, out_hbm, *rest):
        srcv = rest[:_G]
        dstv = rest[_G:2 * _G]
        rows = rest[2 * _G:3 * _G]
        acc = rest[3 * _G]
        semg = rest[3 * _G + 1]
        c = lax.axis_index("c")
        s = lax.axis_index("s")
        base = (c * _NS + s) * epw
        _striped_copy(n, s, lambda sl: pltpu.sync_copy(
            hs_hbm.at[sl], acc.at[sl]))
        plsc.subcore_barrier()

        @pl.loop(0, ngrp)
        def _(t):
            g0 = base + t * _G * _K
            cps = []
            for i in range(_G):
                pltpu.sync_copy(src_hbm.at[pl.ds(g0 + i * _K, _K)], srcv[i])
                pltpu.sync_copy(dst_hbm.at[pl.ds(g0 + i * _K, _K)], dstv[i])
                cps.append(pltpu.async_copy(hs_hbm.at[srcv[i]], rows[i], semg))
            for cp in cps:
                cp.wait()
            for i in range(_G):
                pltpu.sync_copy(rows[i], acc.at[dstv[i]], add=True)

        plsc.subcore_barrier()
        _striped_copy(n, s, lambda sl: pltpu.sync_copy(
            acc.at[sl], out_hbm.at[c].at[sl]))

    return jax.jit(agg_kernel)


def _sc_aggregate(hs, src, dst):
    n, h = hs.shape
    return _make_sc_aggregate(n, h, src.shape[0])(hs, src, dst)


def _tc_matmul(x, w):
    n = x.shape[0]
    h = w.shape[1]

    def body(x_ref, w_ref, o_ref):
        o_ref[...] = jnp.dot(x_ref[...], w_ref[...],
                             preferred_element_type=jnp.float32)

    return pl.pallas_call(
        body, out_shape=jax.ShapeDtypeStruct((n, h), jnp.float32))(x, w)


def _tc_scale(h1, degp):
    n, h = h1.shape

    def body(h_ref, deg_ref, hs_ref, dinv_ref):
        deg = deg_ref[0] + deg_ref[1]
        dinv = lax.rsqrt(deg)
        dcol = dinv[:, 0:1]
        hs_ref[...] = h_ref[...] * dcol
        dinv_ref[...] = dcol

    return pl.pallas_call(
        body,
        out_shape=[
            jax.ShapeDtypeStruct((n, h), jnp.float32),
            jax.ShapeDtypeStruct((n, 1), jnp.float32),
        ],
    )(h1, degp)


def _tc_mid(p, hs, dinv, b, g, be, w_next):
    n, h = hs.shape

    def body(p_ref, hs_ref, dinv_ref, b_ref, g_ref, be_ref, w_ref, o_ref):
        dcol = dinv_ref[...]
        z = dcol * (p_ref[0] + p_ref[1] - hs_ref[...]) + b_ref[...][None, :]
        mean = jnp.mean(z, axis=0, keepdims=True)
        zc = z - mean
        var = jnp.mean(zc * zc, axis=0, keepdims=True)
        zn = g_ref[...][None, :] * zc * lax.rsqrt(var + 1e-5) + be_ref[...][None, :]
        a = jnp.maximum(zn, 0.0)
        o_ref[...] = jnp.dot(a, w_ref[...], preferred_element_type=jnp.float32) * dcol

    return pl.pallas_call(
        body,
        out_shape=jax.ShapeDtypeStruct((n, h), jnp.float32),
    )(p, hs, dinv, b, g, be, w_next)


def _tc_fin(p, hs, dinv, b):
    n, h = hs.shape

    def body(p_ref, hs_ref, dinv_ref, b_ref, o_ref):
        dcol = dinv_ref[...]
        o_ref[...] = dcol * (p_ref[0] + p_ref[1] - hs_ref[...]) + b_ref[...][None, :]

    return pl.pallas_call(
        body,
        out_shape=jax.ShapeDtypeStruct((n, h), jnp.float32),
    )(p, hs, dinv, b)


def kernel(x, edge_index, W1, b1, W2, b2, W3, b3, g1, be1, g2, be2):
    n = x.shape[0]
    e = edge_index.shape[1]
    src = edge_index[0]
    dst = edge_index[1]
    init = jnp.concatenate(
        [jnp.ones((1, n, _DW), jnp.float32), jnp.zeros((1, n, _DW), jnp.float32)]
    )
    h1 = _tc_matmul(x, W1)
    degp = _sc_degree(dst, init)
    hs1, dinv = _tc_scale(h1, degp)
    p1 = _sc_aggregate(hs1, src, dst)
    hs2 = _tc_mid(p1, hs1, dinv, b1, g1, be1, W2)
    p2 = _sc_aggregate(hs2, src, dst)
    hs3 = _tc_mid(p2, hs2, dinv, b2, g2, be2, W3)
    p3 = _sc_aggregate(hs3, src, dst)
    return _tc_fin(p3, hs3, dinv, b3)


# speedup vs baseline: 1.4129x; 1.4129x over previous
import functools

import jax
import jax.numpy as jnp
from jax import lax
from jax.experimental import pallas as pl
from jax.experimental.pallas import tpu as pltpu
from jax.experimental.pallas import tpu_sc as plsc

_NC = 2
_NS = 16
_DW = 16
_K = 40
_G = 5


def _mesh():
    return plsc.VectorSubcoreMesh(
        core_axis_name="c", subcore_axis_name="s",
        num_cores=_NC, num_subcores=_NS)


def _striped_copy(n, s, copy_fn):
    rpt8 = (n // _NS) // 8 * 8
    rem = n - _NS * rpt8
    copy_fn(pl.ds(s * rpt8, rpt8))
    if rem:
        @pl.when(s == _NS - 1)
        def _():
            copy_fn(pl.ds(_NS * rpt8, rem))


def _sc_degree(dst, init):
    e = dst.shape[0]
    n = init.shape[1]
    nw = _NC * _NS
    epw = e // nw
    nwin = epw // _K

    @functools.partial(
        pl.kernel,
        out_type=jax.ShapeDtypeStruct((_NC, n, _DW), jnp.float32),
        mesh=_mesh(),
        scratch_types=[
            pltpu.VMEM((_K,), jnp.int32),
            pltpu.VMEM((_K, _DW), jnp.float32),
            pltpu.VMEM_SHARED((n, _DW), jnp.float32),
        ],
    )
    def deg_kernel(dst_hbm, init_hbm, out_hbm, dstv, ones, acc):
        c = lax.axis_index("c")
        s = lax.axis_index("s")
        base = (c * _NS + s) * epw

        @pl.loop(0, _K)
        def _(i):
            ones[i, :] = jnp.full((_DW,), 1.0, jnp.float32)

        _striped_copy(n, s, lambda sl: pltpu.sync_copy(
            init_hbm.at[c].at[sl], acc.at[sl]))
        plsc.subcore_barrier()

        @pl.loop(0, nwin)
        def _(t):
            pltpu.sync_copy(dst_hbm.at[pl.ds(base + t * _K, _K)], dstv)
            pltpu.sync_copy(ones, acc.at[dstv], add=True)

        plsc.subcore_barrier()
        _striped_copy(n, s, lambda sl: pltpu.sync_copy(
            acc.at[sl], out_hbm.at[c].at[sl]))

    return deg_kernel(dst, init)


@functools.lru_cache(maxsize=None)
def _make_sc_aggregate(n, h, e):
    nw = _NC * _NS
    epw = e // nw
    ngrp = epw // (_G * _K)

    @functools.partial(
        pl.kernel,
        out_type=jax.ShapeDtypeStruct((_NC, n, h), jnp.float32),
        mesh=_mesh(),
        scratch_types=[
            pltpu.VMEM((_G * _K,), jnp.int32),
            pltpu.VMEM((_G * _K,), jnp.int32),
        ]
        + [pltpu.VMEM((_K, h), jnp.float32)] * _G
        + [
            pltpu.VMEM_SHARED((n, h), jnp.float32),
            pltpu.SemaphoreType.DMA,
        ],
    )
    def agg_kernel(hs_hbm, src_hbm, dst_hbm, out_hbm, srcb, dstb, *rest):
        rows = rest[:_G]
        acc = rest[_G]
        semg = rest[_G + 1]
        c = lax.axis_index("c")
        s = lax.axis_index("s")
        base = (c * _NS + s) * epw
        _striped_copy(n, s, lambda sl: pltpu.sync_copy(
            hs_hbm.at[sl], acc.at[sl]))
        plsc.subcore_barrier()

        @pl.loop(0, ngrp)
        def _(t):
            g0 = base + t * _G * _K
            pltpu.sync_copy(src_hbm.at[pl.ds(g0, _G * _K)], srcb)
            pltpu.sync_copy(dst_hbm.at[pl.ds(g0, _G * _K)], dstb)
            cps = [pltpu.async_copy(
                       hs_hbm.at[srcb.at[pl.ds(i * _K, _K)]], rows[i], semg)
                   for i in range(_G)]
            for cp in cps:
                cp.wait()
            for i in range(_G):
                pltpu.sync_copy(rows[i], acc.at[dstb.at[pl.ds(i * _K, _K)]],
                                add=True)

        plsc.subcore_barrier()
        _striped_copy(n, s, lambda sl: pltpu.sync_copy(
            acc.at[sl], out_hbm.at[c].at[sl]))

    return jax.jit(agg_kernel)


def _sc_aggregate(hs, src, dst):
    n, h = hs.shape
    return _make_sc_aggregate(n, h, src.shape[0])(hs, src, dst)


def _tc_matmul(x, w):
    n = x.shape[0]
    h = w.shape[1]

    def body(x_ref, w_ref, o_ref):
        o_ref[...] = jnp.dot(x_ref[...], w_ref[...],
                             preferred_element_type=jnp.float32)

    return pl.pallas_call(
        body, out_shape=jax.ShapeDtypeStruct((n, h), jnp.float32))(x, w)


def _tc_scale(h1, degp):
    n, h = h1.shape

    def body(h_ref, deg_ref, hs_ref, dinv_ref):
        deg = deg_ref[0] + deg_ref[1]
        dinv = lax.rsqrt(deg)
        dcol = dinv[:, 0:1]
        hs_ref[...] = h_ref[...] * dcol
        dinv_ref[...] = dcol

    return pl.pallas_call(
        body,
        out_shape=[
            jax.ShapeDtypeStruct((n, h), jnp.float32),
            jax.ShapeDtypeStruct((n, 1), jnp.float32),
        ],
    )(h1, degp)


def _tc_mid(p, hs, dinv, b, g, be, w_next):
    n, h = hs.shape

    def body(p_ref, hs_ref, dinv_ref, b_ref, g_ref, be_ref, w_ref, o_ref):
        dcol = dinv_ref[...]
        z = dcol * (p_ref[0] + p_ref[1] - hs_ref[...]) + b_ref[...][None, :]
        mean = jnp.mean(z, axis=0, keepdims=True)
        zc = z - mean
        var = jnp.mean(zc * zc, axis=0, keepdims=True)
        zn = g_ref[...][None, :] * zc * lax.rsqrt(var + 1e-5) + be_ref[...][None, :]
        a = jnp.maximum(zn, 0.0)
        o_ref[...] = jnp.dot(a, w_ref[...], preferred_element_type=jnp.float32) * dcol

    return pl.pallas_call(
        body,
        out_shape=jax.ShapeDtypeStruct((n, h), jnp.float32),
    )(p, hs, dinv, b, g, be, w_next)


def _tc_fin(p, hs, dinv, b):
    n, h = hs.shape

    def body(p_ref, hs_ref, dinv_ref, b_ref, o_ref):
        dcol = dinv_ref[...]
        o_ref[...] = dcol * (p_ref[0] + p_ref[1] - hs_ref[...]) + b_ref[...][None, :]

    return pl.pallas_call(
        body,
        out_shape=jax.ShapeDtypeStruct((n, h), jnp.float32),
    )(p, hs, dinv, b)


def kernel(x, edge_index, W1, b1, W2, b2, W3, b3, g1, be1, g2, be2):
    n = x.shape[0]
    e = edge_index.shape[1]
    src = edge_index[0]
    dst = edge_index[1]
    init = jnp.concatenate(
        [jnp.ones((1, n, _DW), jnp.float32), jnp.zeros((1, n, _DW), jnp.float32)]
    )
    h1 = _tc_matmul(x, W1)
    degp = _sc_degree(dst, init)
    hs1, dinv = _tc_scale(h1, degp)
    p1 = _sc_aggregate(hs1, src, dst)
    hs2 = _tc_mid(p1, hs1, dinv, b1, g1, be1, W2)
    p2 = _sc_aggregate(hs2, src, dst)
    hs3 = _tc_mid(p2, hs2, dinv, b2, g2, be2, W3)
    p3 = _sc_aggregate(hs3, src, dst)
    return _tc_fin(p3, hs3, dinv, b3)


# speedup vs baseline: 1.6215x; 1.1476x over previous
import functools

import jax
import jax.numpy as jnp
from jax import lax
from jax.experimental import pallas as pl
from jax.experimental.pallas import tpu as pltpu
from jax.experimental.pallas import tpu_sc as plsc

_NC = 2
_NS = 16
_DW = 16
_K = 40
_G = 5


def _mesh():
    return plsc.VectorSubcoreMesh(
        core_axis_name="c", subcore_axis_name="s",
        num_cores=_NC, num_subcores=_NS)


def _striped_copy(n, s, copy_fn):
    rpt8 = (n // _NS) // 8 * 8
    rem = n - _NS * rpt8
    copy_fn(pl.ds(s * rpt8, rpt8))
    if rem:
        @pl.when(s == _NS - 1)
        def _():
            copy_fn(pl.ds(_NS * rpt8, rem))


def _sc_degree(dst, init):
    e = dst.shape[0]
    n = init.shape[1]
    nw = _NC * _NS
    epw = e // nw
    nwin = epw // _K

    @functools.partial(
        pl.kernel,
        out_type=jax.ShapeDtypeStruct((_NC, n, _DW), jnp.float32),
        mesh=_mesh(),
        scratch_types=[
            pltpu.VMEM((_K,), jnp.int32),
            pltpu.VMEM((_K, _DW), jnp.float32),
            pltpu.VMEM_SHARED((n, _DW), jnp.float32),
        ],
    )
    def deg_kernel(dst_hbm, init_hbm, out_hbm, dstv, ones, acc):
        c = lax.axis_index("c")
        s = lax.axis_index("s")
        base = (c * _NS + s) * epw

        @pl.loop(0, _K)
        def _(i):
            ones[i, :] = jnp.full((_DW,), 1.0, jnp.float32)

        _striped_copy(n, s, lambda sl: pltpu.sync_copy(
            init_hbm.at[c].at[sl], acc.at[sl]))
        plsc.subcore_barrier()

        @pl.loop(0, nwin)
        def _(t):
            pltpu.sync_copy(dst_hbm.at[pl.ds(base + t * _K, _K)], dstv)
            pltpu.sync_copy(ones, acc.at[dstv], add=True)

        plsc.subcore_barrier()
        _striped_copy(n, s, lambda sl: pltpu.sync_copy(
            acc.at[sl], out_hbm.at[c].at[sl]))

    return deg_kernel(dst, init)


@functools.lru_cache(maxsize=None)
def _make_sc_aggregate(n, h, e):
    nw = _NC * _NS
    epw = e // nw
    ngrp = epw // (_G * _K)
    npair = ngrp // 2

    @functools.partial(
        pl.kernel,
        out_type=jax.ShapeDtypeStruct((_NC, n, h), jnp.float32),
        mesh=_mesh(),
        scratch_types=[pltpu.VMEM((_G * _K,), jnp.int32)] * 4
        + [pltpu.VMEM((_K, h), jnp.float32)] * _G
        + [pltpu.VMEM_SHARED((n, h), jnp.float32)]
        + [pltpu.SemaphoreType.DMA] * _G,
    )
    def agg_kernel(hs_hbm, src_hbm, dst_hbm, out_hbm, *rest):
        srca, dsta, srcb, dstb = rest[:4]
        rows = rest[4:4 + _G]
        acc = rest[4 + _G]
        sems = rest[5 + _G:]
        c = lax.axis_index("c")
        s = lax.axis_index("s")
        base = (c * _NS + s) * epw
        gk = _G * _K

        def load_idx(t, sref, dref):
            pltpu.sync_copy(src_hbm.at[pl.ds(base + t * gk, gk)], sref)
            pltpu.sync_copy(dst_hbm.at[pl.ds(base + t * gk, gk)], dref)

        def fire(sref):
            return [pltpu.async_copy(
                        hs_hbm.at[sref.at[pl.ds(i * _K, _K)]], rows[i],
                        sems[i])
                    for i in range(_G)]

        def drain_scatter(cps, dref):
            for i in range(_G):
                cps[i].wait()
                pltpu.sync_copy(rows[i], acc.at[dref.at[pl.ds(i * _K, _K)]],
                                add=True)

        _striped_copy(n, s, lambda sl: pltpu.sync_copy(
            hs_hbm.at[sl], acc.at[sl]))
        load_idx(0, srca, dsta)
        plsc.subcore_barrier()

        @pl.loop(0, npair)
        def _(u):
            cps = fire(srca)
            load_idx(2 * u + 1, srcb, dstb)
            drain_scatter(cps, dsta)
            cps = fire(srcb)

            @pl.when(u + 1 < npair)
            def _():
                load_idx(2 * u + 2, srca, dsta)

            drain_scatter(cps, dstb)

        plsc.subcore_barrier()
        _striped_copy(n, s, lambda sl: pltpu.sync_copy(
            acc.at[sl], out_hbm.at[c].at[sl]))

    return jax.jit(agg_kernel)


def _sc_aggregate(hs, src, dst):
    n, h = hs.shape
    return _make_sc_aggregate(n, h, src.shape[0])(hs, src, dst)


def _tc_matmul(x, w):
    n = x.shape[0]
    h = w.shape[1]

    def body(x_ref, w_ref, o_ref):
        o_ref[...] = jnp.dot(x_ref[...], w_ref[...],
                             preferred_element_type=jnp.float32)

    return pl.pallas_call(
        body, out_shape=jax.ShapeDtypeStruct((n, h), jnp.float32))(x, w)


def _tc_scale(h1, degp):
    n, h = h1.shape

    def body(h_ref, deg_ref, hs_ref, dinv_ref):
        deg = deg_ref[0] + deg_ref[1]
        dinv = lax.rsqrt(deg)
        dcol = dinv[:, 0:1]
        hs_ref[...] = h_ref[...] * dcol
        dinv_ref[...] = dcol

    return pl.pallas_call(
        body,
        out_shape=[
            jax.ShapeDtypeStruct((n, h), jnp.float32),
            jax.ShapeDtypeStruct((n, 1), jnp.float32),
        ],
    )(h1, degp)


def _tc_mid(p, hs, dinv, b, g, be, w_next):
    n, h = hs.shape

    def body(p_ref, hs_ref, dinv_ref, b_ref, g_ref, be_ref, w_ref, o_ref):
        dcol = dinv_ref[...]
        z = dcol * (p_ref[0] + p_ref[1] - hs_ref[...]) + b_ref[...][None, :]
        mean = jnp.mean(z, axis=0, keepdims=True)
        zc = z - mean
        var = jnp.mean(zc * zc, axis=0, keepdims=True)
        zn = g_ref[...][None, :] * zc * lax.rsqrt(var + 1e-5) + be_ref[...][None, :]
        a = jnp.maximum(zn, 0.0)
        o_ref[...] = jnp.dot(a, w_ref[...], preferred_element_type=jnp.float32) * dcol

    return pl.pallas_call(
        body,
        out_shape=jax.ShapeDtypeStruct((n, h), jnp.float32),
    )(p, hs, dinv, b, g, be, w_next)


def _tc_fin(p, hs, dinv, b):
    n, h = hs.shape

    def body(p_ref, hs_ref, dinv_ref, b_ref, o_ref):
        dcol = dinv_ref[...]
        o_ref[...] = dcol * (p_ref[0] + p_ref[1] - hs_ref[...]) + b_ref[...][None, :]

    return pl.pallas_call(
        body,
        out_shape=jax.ShapeDtypeStruct((n, h), jnp.float32),
    )(p, hs, dinv, b)


def kernel(x, edge_index, W1, b1, W2, b2, W3, b3, g1, be1, g2, be2):
    n = x.shape[0]
    e = edge_index.shape[1]
    src = edge_index[0]
    dst = edge_index[1]
    init = jnp.concatenate(
        [jnp.ones((1, n, _DW), jnp.float32), jnp.zeros((1, n, _DW), jnp.float32)]
    )
    h1 = _tc_matmul(x, W1)
    degp = _sc_degree(dst, init)
    hs1, dinv = _tc_scale(h1, degp)
    p1 = _sc_aggregate(hs1, src, dst)
    hs2 = _tc_mid(p1, hs1, dinv, b1, g1, be1, W2)
    p2 = _sc_aggregate(hs2, src, dst)
    hs3 = _tc_mid(p2, hs2, dinv, b2, g2, be2, W3)
    p3 = _sc_aggregate(hs3, src, dst)
    return _tc_fin(p3, hs3, dinv, b3)


# speedup vs baseline: 1.7427x; 1.0747x over previous
import functools

import jax
import jax.numpy as jnp
from jax import lax
from jax.experimental import pallas as pl
from jax.experimental.pallas import tpu as pltpu
from jax.experimental.pallas import tpu_sc as plsc

_NC = 2
_NS = 16
_DW = 16
_K = 40
_G = 5


def _mesh():
    return plsc.VectorSubcoreMesh(
        core_axis_name="c", subcore_axis_name="s",
        num_cores=_NC, num_subcores=_NS)


def _striped_copy(n, s, copy_fn):
    rpt8 = (n // _NS) // 8 * 8
    rem = n - _NS * rpt8
    copy_fn(pl.ds(s * rpt8, rpt8))
    if rem:
        @pl.when(s == _NS - 1)
        def _():
            copy_fn(pl.ds(_NS * rpt8, rem))


def _sc_degree(dst, init):
    e = dst.shape[0]
    n = init.shape[1]
    nw = _NC * _NS
    epw = e // nw
    nwin = epw // _K

    @functools.partial(
        pl.kernel,
        out_type=jax.ShapeDtypeStruct((_NC, n, _DW), jnp.float32),
        mesh=_mesh(),
        scratch_types=[
            pltpu.VMEM((_K,), jnp.int32),
            pltpu.VMEM((_K, _DW), jnp.float32),
            pltpu.VMEM_SHARED((n, _DW), jnp.float32),
        ],
    )
    def deg_kernel(dst_hbm, init_hbm, out_hbm, dstv, ones, acc):
        c = lax.axis_index("c")
        s = lax.axis_index("s")
        base = (c * _NS + s) * epw

        @pl.loop(0, _K)
        def _(i):
            ones[i, :] = jnp.full((_DW,), 1.0, jnp.float32)

        _striped_copy(n, s, lambda sl: pltpu.sync_copy(
            init_hbm.at[c].at[sl], acc.at[sl]))
        plsc.subcore_barrier()

        @pl.loop(0, nwin)
        def _(t):
            pltpu.sync_copy(dst_hbm.at[pl.ds(base + t * _K, _K)], dstv)
            pltpu.sync_copy(ones, acc.at[dstv], add=True)

        plsc.subcore_barrier()
        _striped_copy(n, s, lambda sl: pltpu.sync_copy(
            acc.at[sl], out_hbm.at[c].at[sl]))

    return deg_kernel(dst, init)


@functools.lru_cache(maxsize=None)
def _make_sc_aggregate(n, h, e):
    nw = _NC * _NS
    epw = e // nw
    ngrp = epw // (_G * _K)
    npair = ngrp // 2

    @functools.partial(
        pl.kernel,
        out_type=jax.ShapeDtypeStruct((_NC, n, h), jnp.float32),
        mesh=_mesh(),
        scratch_types=[pltpu.VMEM((_G * _K,), jnp.int32)] * 4
        + [pltpu.VMEM((_K, h), jnp.float32)] * _G
        + [pltpu.VMEM_SHARED((n, h), jnp.float32)]
        + [pltpu.SemaphoreType.DMA] * (_G + 2),
    )
    def agg_kernel(hs_hbm, src_hbm, dst_hbm, out_hbm, *rest):
        srca, dsta, srcb, dstb = rest[:4]
        rows = rest[4:4 + _G]
        acc = rest[4 + _G]
        sems = rest[5 + _G:5 + 2 * _G]
        semia, semib = rest[5 + 2 * _G:]
        c = lax.axis_index("c")
        s = lax.axis_index("s")
        base = (c * _NS + s) * epw
        gk = _G * _K

        def idx_slices(t):
            sl = pl.ds(base + t * gk, gk)
            return src_hbm.at[sl], dst_hbm.at[sl]

        def load_idx(t, sref, dref, sem):
            shbm, dhbm = idx_slices(t)
            pltpu.async_copy(shbm, sref, sem)
            pltpu.async_copy(dhbm, dref, sem)

        def wait_idx(t, sref, dref, sem):
            shbm, dhbm = idx_slices(t)
            pltpu.make_async_copy(shbm, sref, sem).wait()
            pltpu.make_async_copy(dhbm, dref, sem).wait()

        def fire(sref):
            return [pltpu.async_copy(
                        hs_hbm.at[sref.at[pl.ds(i * _K, _K)]], rows[i],
                        sems[i])
                    for i in range(_G)]

        def drain_scatter(cps, dref):
            for i in range(_G):
                cps[i].wait()
                pltpu.sync_copy(rows[i], acc.at[dref.at[pl.ds(i * _K, _K)]],
                                add=True)

        _striped_copy(n, s, lambda sl: pltpu.sync_copy(
            hs_hbm.at[sl], acc.at[sl]))
        load_idx(0, srca, dsta, semia)
        plsc.subcore_barrier()

        @pl.loop(0, npair - 1)
        def _(u):
            wait_idx(2 * u, srca, dsta, semia)
            cps = fire(srca)
            load_idx(2 * u + 1, srcb, dstb, semib)
            drain_scatter(cps, dsta)
            wait_idx(2 * u + 1, srcb, dstb, semib)
            cps = fire(srcb)
            load_idx(2 * u + 2, srca, dsta, semia)
            drain_scatter(cps, dstb)

        wait_idx(ngrp - 2, srca, dsta, semia)
        cps = fire(srca)
        load_idx(ngrp - 1, srcb, dstb, semib)
        drain_scatter(cps, dsta)
        wait_idx(ngrp - 1, srcb, dstb, semib)
        cps = fire(srcb)
        drain_scatter(cps, dstb)

        plsc.subcore_barrier()
        _striped_copy(n, s, lambda sl: pltpu.sync_copy(
            acc.at[sl], out_hbm.at[c].at[sl]))

    return jax.jit(agg_kernel)


def _sc_aggregate(hs, src, dst):
    n, h = hs.shape
    return _make_sc_aggregate(n, h, src.shape[0])(hs, src, dst)


def _tc_matmul(x, w):
    n = x.shape[0]
    h = w.shape[1]

    def body(x_ref, w_ref, o_ref):
        o_ref[...] = jnp.dot(x_ref[...], w_ref[...],
                             preferred_element_type=jnp.float32)

    return pl.pallas_call(
        body, out_shape=jax.ShapeDtypeStruct((n, h), jnp.float32))(x, w)


def _tc_scale(h1, degp):
    n, h = h1.shape

    def body(h_ref, deg_ref, hs_ref, dinv_ref):
        deg = deg_ref[0] + deg_ref[1]
        dinv = lax.rsqrt(deg)
        dcol = dinv[:, 0:1]
        hs_ref[...] = h_ref[...] * dcol
        dinv_ref[...] = dcol

    return pl.pallas_call(
        body,
        out_shape=[
            jax.ShapeDtypeStruct((n, h), jnp.float32),
            jax.ShapeDtypeStruct((n, 1), jnp.float32),
        ],
    )(h1, degp)


def _tc_mid(p, hs, dinv, b, g, be, w_next):
    n, h = hs.shape

    def body(p_ref, hs_ref, dinv_ref, b_ref, g_ref, be_ref, w_ref, o_ref):
        dcol = dinv_ref[...]
        z = dcol * (p_ref[0] + p_ref[1] - hs_ref[...]) + b_ref[...][None, :]
        mean = jnp.mean(z, axis=0, keepdims=True)
        zc = z - mean
        var = jnp.mean(zc * zc, axis=0, keepdims=True)
        zn = g_ref[...][None, :] * zc * lax.rsqrt(var + 1e-5) + be_ref[...][None, :]
        a = jnp.maximum(zn, 0.0)
        o_ref[...] = jnp.dot(a, w_ref[...], preferred_element_type=jnp.float32) * dcol

    return pl.pallas_call(
        body,
        out_shape=jax.ShapeDtypeStruct((n, h), jnp.float32),
    )(p, hs, dinv, b, g, be, w_next)


def _tc_fin(p, hs, dinv, b):
    n, h = hs.shape

    def body(p_ref, hs_ref, dinv_ref, b_ref, o_ref):
        dcol = dinv_ref[...]
        o_ref[...] = dcol * (p_ref[0] + p_ref[1] - hs_ref[...]) + b_ref[...][None, :]

    return pl.pallas_call(
        body,
        out_shape=jax.ShapeDtypeStruct((n, h), jnp.float32),
    )(p, hs, dinv, b)


def kernel(x, edge_index, W1, b1, W2, b2, W3, b3, g1, be1, g2, be2):
    n = x.shape[0]
    e = edge_index.shape[1]
    src = edge_index[0]
    dst = edge_index[1]
    init = jnp.concatenate(
        [jnp.ones((1, n, _DW), jnp.float32), jnp.zeros((1, n, _DW), jnp.float32)]
    )
    h1 = _tc_matmul(x, W1)
    degp = _sc_degree(dst, init)
    hs1, dinv = _tc_scale(h1, degp)
    p1 = _sc_aggregate(hs1, src, dst)
    hs2 = _tc_mid(p1, hs1, dinv, b1, g1, be1, W2)
    p2 = _sc_aggregate(hs2, src, dst)
    hs3 = _tc_mid(p2, hs2, dinv, b2, g2, be2, W3)
    p3 = _sc_aggregate(hs3, src, dst)
    return _tc_fin(p3, hs3, dinv, b3)


# speedup vs baseline: 2.1729x; 1.2468x over previous
import functools

import jax
import jax.numpy as jnp
from jax import lax
from jax.experimental import pallas as pl
from jax.experimental.pallas import tpu as pltpu
from jax.experimental.pallas import tpu_sc as plsc

_NC = 2
_NS = 16
_DW = 16
_K = 40
_G = 5


def _mesh():
    return plsc.VectorSubcoreMesh(
        core_axis_name="c", subcore_axis_name="s",
        num_cores=_NC, num_subcores=_NS)


def _striped_copy(n, s, copy_fn):
    rpt8 = (n // _NS) // 8 * 8
    rem = n - _NS * rpt8
    copy_fn(pl.ds(s * rpt8, rpt8))
    if rem:
        @pl.when(s == _NS - 1)
        def _():
            copy_fn(pl.ds(_NS * rpt8, rem))


def _sc_degree(dst, init):
    e = dst.shape[0]
    n = init.shape[1]
    nw = _NC * _NS
    epw = e // nw
    gk = _G * _K
    ngrp = epw // gk
    npair = ngrp // 2

    @functools.partial(
        pl.kernel,
        out_type=jax.ShapeDtypeStruct((_NC, n, _DW), jnp.float32),
        mesh=_mesh(),
        scratch_types=[
            pltpu.VMEM((gk,), jnp.int32),
            pltpu.VMEM((gk,), jnp.int32),
            pltpu.VMEM((_K, _DW), jnp.float32),
            pltpu.VMEM_SHARED((n, _DW), jnp.float32),
        ] + [pltpu.SemaphoreType.DMA] * (_G + 2),
    )
    def deg_kernel(dst_hbm, init_hbm, out_hbm, dsta, dstb, ones, acc, *sems):
        semsc = sems[:_G]
        semia, semib = sems[_G:]
        c = lax.axis_index("c")
        s = lax.axis_index("s")
        base = (c * _NS + s) * epw

        def idx_slice(t):
            return dst_hbm.at[pl.ds(base + t * gk, gk)]

        def scatter_group(dref):
            scs = [pltpu.async_copy(
                       ones, acc.at[dref.at[pl.ds(i * _K, _K)]], semsc[i],
                       add=True)
                   for i in range(_G)]
            for d in scs:
                d.wait()

        @pl.loop(0, _K)
        def _(i):
            ones[i, :] = jnp.full((_DW,), 1.0, jnp.float32)

        _striped_copy(n, s, lambda sl: pltpu.sync_copy(
            init_hbm.at[c].at[sl], acc.at[sl]))
        pltpu.async_copy(idx_slice(0), dsta, semia)
        plsc.subcore_barrier()

        @pl.loop(0, npair - 1)
        def _(u):
            pltpu.make_async_copy(idx_slice(2 * u), dsta, semia).wait()
            pltpu.async_copy(idx_slice(2 * u + 1), dstb, semib)
            scatter_group(dsta)
            pltpu.make_async_copy(idx_slice(2 * u + 1), dstb, semib).wait()
            pltpu.async_copy(idx_slice(2 * u + 2), dsta, semia)
            scatter_group(dstb)

        pltpu.make_async_copy(idx_slice(ngrp - 2), dsta, semia).wait()
        pltpu.async_copy(idx_slice(ngrp - 1), dstb, semib)
        scatter_group(dsta)
        pltpu.make_async_copy(idx_slice(ngrp - 1), dstb, semib).wait()
        scatter_group(dstb)

        plsc.subcore_barrier()
        _striped_copy(n, s, lambda sl: pltpu.sync_copy(
            acc.at[sl], out_hbm.at[c].at[sl]))

    return deg_kernel(dst, init)


@functools.lru_cache(maxsize=None)
def _make_sc_aggregate(n, h, e):
    nw = _NC * _NS
    epw = e // nw
    ngrp = epw // (_G * _K)
    npair = ngrp // 2

    @functools.partial(
        pl.kernel,
        out_type=jax.ShapeDtypeStruct((_NC, n, h), jnp.float32),
        mesh=_mesh(),
        scratch_types=[pltpu.VMEM((_G * _K,), jnp.int32)] * 4
        + [pltpu.VMEM((_K, h), jnp.float32)] * _G
        + [pltpu.VMEM_SHARED((n, h), jnp.float32)]
        + [pltpu.SemaphoreType.DMA] * (2 * _G + 2),
    )
    def agg_kernel(hs_hbm, src_hbm, dst_hbm, out_hbm, *rest):
        srca, dsta, srcb, dstb = rest[:4]
        rows = rest[4:4 + _G]
        acc = rest[4 + _G]
        sems = rest[5 + _G:5 + 2 * _G]
        semsc = rest[5 + 2 * _G:5 + 3 * _G]
        semia, semib = rest[5 + 3 * _G:]
        c = lax.axis_index("c")
        s = lax.axis_index("s")
        base = (c * _NS + s) * epw
        gk = _G * _K

        def idx_slices(t):
            sl = pl.ds(base + t * gk, gk)
            return src_hbm.at[sl], dst_hbm.at[sl]

        def load_idx(t, sref, dref, sem):
            shbm, dhbm = idx_slices(t)
            pltpu.async_copy(shbm, sref, sem)
            pltpu.async_copy(dhbm, dref, sem)

        def wait_idx(t, sref, dref, sem):
            shbm, dhbm = idx_slices(t)
            pltpu.make_async_copy(shbm, sref, sem).wait()
            pltpu.make_async_copy(dhbm, dref, sem).wait()

        def fire(sref):
            return [pltpu.async_copy(
                        hs_hbm.at[sref.at[pl.ds(i * _K, _K)]], rows[i],
                        sems[i])
                    for i in range(_G)]

        def drain_scatter(cps, dref):
            scs = []
            for i in range(_G):
                cps[i].wait()
                scs.append(pltpu.async_copy(
                    rows[i], acc.at[dref.at[pl.ds(i * _K, _K)]], semsc[i],
                    add=True))
            for d in scs:
                d.wait()

        _striped_copy(n, s, lambda sl: pltpu.sync_copy(
            hs_hbm.at[sl], acc.at[sl]))
        load_idx(0, srca, dsta, semia)
        plsc.subcore_barrier()

        @pl.loop(0, npair - 1)
        def _(u):
            wait_idx(2 * u, srca, dsta, semia)
            cps = fire(srca)
            load_idx(2 * u + 1, srcb, dstb, semib)
            drain_scatter(cps, dsta)
            wait_idx(2 * u + 1, srcb, dstb, semib)
            cps = fire(srcb)
            load_idx(2 * u + 2, srca, dsta, semia)
            drain_scatter(cps, dstb)

        wait_idx(ngrp - 2, srca, dsta, semia)
        cps = fire(srca)
        load_idx(ngrp - 1, srcb, dstb, semib)
        drain_scatter(cps, dsta)
        wait_idx(ngrp - 1, srcb, dstb, semib)
        cps = fire(srcb)
        drain_scatter(cps, dstb)

        plsc.subcore_barrier()
        _striped_copy(n, s, lambda sl: pltpu.sync_copy(
            acc.at[sl], out_hbm.at[c].at[sl]))

    return jax.jit(agg_kernel)


def _sc_aggregate(hs, src, dst):
    n, h = hs.shape
    return _make_sc_aggregate(n, h, src.shape[0])(hs, src, dst)


def _tc_matmul(x, w):
    n = x.shape[0]
    h = w.shape[1]

    def body(x_ref, w_ref, o_ref):
        o_ref[...] = jnp.dot(x_ref[...], w_ref[...],
                             preferred_element_type=jnp.float32)

    return pl.pallas_call(
        body, out_shape=jax.ShapeDtypeStruct((n, h), jnp.float32))(x, w)


def _tc_scale(h1, degp):
    n, h = h1.shape

    def body(h_ref, deg_ref, hs_ref, dinv_ref):
        deg = deg_ref[0] + deg_ref[1]
        dinv = lax.rsqrt(deg)
        dcol = dinv[:, 0:1]
        hs_ref[...] = h_ref[...] * dcol
        dinv_ref[...] = dcol

    return pl.pallas_call(
        body,
        out_shape=[
            jax.ShapeDtypeStruct((n, h), jnp.float32),
            jax.ShapeDtypeStruct((n, 1), jnp.float32),
        ],
    )(h1, degp)


def _tc_mid(p, hs, dinv, b, g, be, w_next):
    n, h = hs.shape

    def body(p_ref, hs_ref, dinv_ref, b_ref, g_ref, be_ref, w_ref, o_ref):
        dcol = dinv_ref[...]
        z = dcol * (p_ref[0] + p_ref[1] - hs_ref[...]) + b_ref[...][None, :]
        mean = jnp.mean(z, axis=0, keepdims=True)
        zc = z - mean
        var = jnp.mean(zc * zc, axis=0, keepdims=True)
        zn = g_ref[...][None, :] * zc * lax.rsqrt(var + 1e-5) + be_ref[...][None, :]
        a = jnp.maximum(zn, 0.0)
        o_ref[...] = jnp.dot(a, w_ref[...], preferred_element_type=jnp.float32) * dcol

    return pl.pallas_call(
        body,
        out_shape=jax.ShapeDtypeStruct((n, h), jnp.float32),
    )(p, hs, dinv, b, g, be, w_next)


def _tc_fin(p, hs, dinv, b):
    n, h = hs.shape

    def body(p_ref, hs_ref, dinv_ref, b_ref, o_ref):
        dcol = dinv_ref[...]
        o_ref[...] = dcol * (p_ref[0] + p_ref[1] - hs_ref[...]) + b_ref[...][None, :]

    return pl.pallas_call(
        body,
        out_shape=jax.ShapeDtypeStruct((n, h), jnp.float32),
    )(p, hs, dinv, b)


def kernel(x, edge_index, W1, b1, W2, b2, W3, b3, g1, be1, g2, be2):
    n = x.shape[0]
    e = edge_index.shape[1]
    src = edge_index[0]
    dst = edge_index[1]
    init = jnp.concatenate(
        [jnp.ones((1, n, _DW), jnp.float32), jnp.zeros((1, n, _DW), jnp.float32)]
    )
    h1 = _tc_matmul(x, W1)
    degp = _sc_degree(dst, init)
    hs1, dinv = _tc_scale(h1, degp)
    p1 = _sc_aggregate(hs1, src, dst)
    hs2 = _tc_mid(p1, hs1, dinv, b1, g1, be1, W2)
    p2 = _sc_aggregate(hs2, src, dst)
    hs3 = _tc_mid(p2, hs2, dinv, b2, g2, be2, W3)
    p3 = _sc_aggregate(hs3, src, dst)
    return _tc_fin(p3, hs3, dinv, b3)
